# baseline (speedup 1.0000x reference)
import functools

import jax
import jax.numpy as jnp
from jax import lax
from jax.experimental import pallas as pl
from jax.experimental.pallas import tpu as pltpu
from jax.experimental.pallas import tpu_sc as plsc

_INFO = plsc.get_sparse_core_info()
_NC = _INFO.num_cores
_NS = _INFO.num_subcores
_NW = _NC * _NS
_CHUNK = 128
_L = 16


def kernel(time, table):
    B = time.shape[0]
    V, D = table.shape
    assert D == _L and (V * D) % 128 == 0 and B % (_NW * _CHUNK) == 0
    pack = 128 // D
    b_per_w = B // _NW
    n_ch = b_per_w // _CHUNK

    table2 = table.reshape(V // pack, 128)
    time3 = time.reshape(_NW, n_ch, _CHUNK)
    mesh = plsc.VectorSubcoreMesh(core_axis_name="c", subcore_axis_name="s")

    @functools.partial(
        pl.kernel,
        mesh=mesh,
        out_type=jax.ShapeDtypeStruct((_NW, b_per_w, D), jnp.float32),
        scratch_types=[
            pltpu.VMEM((n_ch, _CHUNK), jnp.int32),
            pltpu.VMEM((n_ch, _CHUNK), jnp.int32),
            pltpu.VMEM((2, _CHUNK, 128), jnp.float32),
            pltpu.VMEM((b_per_w, D), jnp.float32),
            pltpu.SemaphoreType.DMA,
            pltpu.SemaphoreType.DMA,
        ],
        compiler_params=pltpu.CompilerParams(needs_layout_passes=False),
    )
    def gather_k(time_hbm, table_hbm, out_hbm, idx_v, q_v, g_v, o_v, s0, s1):
        wid = lax.axis_index("s") * _NC + lax.axis_index("c")
        sems = (s0, s1)
        pltpu.sync_copy(time_hbm.at[wid], idx_v)
        iota = lax.iota(jnp.int32, _L)
        ngrp = _CHUNK // _L
        for c in range(n_ch):
            for t in range(ngrp):
                v = idx_v[c, pl.ds(t * _L, _L)]
                q_v[c, pl.ds(t * _L, _L)] = lax.shift_right_logical(v, 3)
        copies = {
            c: pltpu.async_copy(table_hbm.at[q_v.at[c]], g_v.at[c], sems[c])
            for c in range(2)
        }
        for c in range(n_ch):
            copies[c].wait()
            for t in range(ngrp):
                rows = iota + (t * _L)
                orows = iota + (c * _CHUNK + t * _L)
                off = lax.shift_left(idx_v[c, pl.ds(t * _L, _L)] & 7, 4)
                for j in range(_L):
                    vals = plsc.load_gather(g_v.at[c % 2], [rows, off + j])
                    plsc.store_scatter(
                        o_v, [orows, jnp.full((_L,), j, jnp.int32)], vals
                    )
            nxt = c + 2
            if nxt < n_ch:
                copies[nxt] = pltpu.async_copy(
                    table_hbm.at[q_v.at[nxt]], g_v.at[nxt % 2], sems[nxt % 2]
                )
        pltpu.sync_copy(o_v, out_hbm.at[wid])

    return gather_k(time3, table2).reshape(B, D)


# speedup vs baseline: 1.0931x; 1.0931x over previous
import functools

import jax
import jax.numpy as jnp
from jax import lax
from jax.experimental import pallas as pl
from jax.experimental.pallas import tpu as pltpu
from jax.experimental.pallas import tpu_sc as plsc

_INFO = plsc.get_sparse_core_info()
_NC = _INFO.num_cores
_NS = _INFO.num_subcores
_NW = _NC * _NS
_CHUNK = 128
_L = 16


def kernel(time, table):
    B = time.shape[0]
    V, D = table.shape
    assert D == _L and B % (_NW * _CHUNK) == 0
    b_per_w = B // _NW
    n_ch = b_per_w // _CHUNK

    time2 = time.reshape(_NW, b_per_w)
    mesh = plsc.VectorSubcoreMesh(core_axis_name="c", subcore_axis_name="s")

    @functools.partial(
        pl.kernel,
        mesh=mesh,
        out_type=jax.ShapeDtypeStruct((B, D), jnp.float32),
        scratch_types=[
            pltpu.VMEM((b_per_w,), jnp.int32),
            pltpu.VMEM((n_ch, _CHUNK), jnp.int32),
            pltpu.VMEM((b_per_w, D), jnp.float32),
            pltpu.SemaphoreType.DMA,
        ],
        compiler_params=pltpu.CompilerParams(use_tc_tiling_on_sc=False),
    )
    def gather_k(time_hbm, table_hbm, out_hbm, idx_v, il_v, rows_v, sem):
        wid = lax.axis_index("s") * _NC + lax.axis_index("c")
        pltpu.sync_copy(time_hbm.at[wid], idx_v)
        ngrp = _CHUNK // _L
        for c in range(n_ch):
            for t in range(ngrp):
                il_v[c, pl.ds(t * _L, _L)] = idx_v[pl.ds(c * _CHUNK + t * _L, _L)]
        copies = [
            pltpu.async_copy(
                table_hbm.at[il_v.at[c]], rows_v.at[pl.ds(c * _CHUNK, _CHUNK)], sem
            )
            for c in range(n_ch)
        ]
        for cpy in copies:
            cpy.wait()
        pltpu.sync_copy(rows_v, out_hbm.at[pl.ds(wid * b_per_w, b_per_w)])

    return gather_k(time2, table)


# speedup vs baseline: 1.0936x; 1.0004x over previous
import functools

import jax
import jax.numpy as jnp
from jax import lax
from jax.experimental import pallas as pl
from jax.experimental.pallas import tpu as pltpu
from jax.experimental.pallas import tpu_sc as plsc

_INFO = plsc.get_sparse_core_info()
_NC = _INFO.num_cores
_NS = _INFO.num_subcores
_NW = _NC * _NS
_CHUNK = 128
_L = 16


def kernel(time, table):
    B = time.shape[0]
    V, D = table.shape
    assert D == _L and B % (_NW * _CHUNK) == 0
    b_per_w = B // _NW
    n_ch = b_per_w // _CHUNK

    mesh = plsc.VectorSubcoreMesh(core_axis_name="c", subcore_axis_name="s")

    @functools.partial(
        pl.kernel,
        mesh=mesh,
        out_type=jax.ShapeDtypeStruct((B, D), jnp.float32),
        scratch_types=[
            pltpu.VMEM((b_per_w,), jnp.int32),
            pltpu.VMEM((n_ch, _CHUNK), jnp.int32),
            pltpu.VMEM((b_per_w, D), jnp.float32),
            pltpu.SemaphoreType.DMA,
        ],
        compiler_params=pltpu.CompilerParams(use_tc_tiling_on_sc=False),
    )
    def gather_k(time_hbm, table_hbm, out_hbm, idx_v, il_v, rows_v, sem):
        wid = lax.axis_index("s") * _NC + lax.axis_index("c")
        pltpu.sync_copy(time_hbm.at[pl.ds(wid * b_per_w, b_per_w)], idx_v)
        ngrp = _CHUNK // _L
        for c in range(n_ch):
            for t in range(ngrp):
                il_v[c, pl.ds(t * _L, _L)] = idx_v[pl.ds(c * _CHUNK + t * _L, _L)]
        copies = [
            pltpu.async_copy(
                table_hbm.at[il_v.at[c]], rows_v.at[pl.ds(c * _CHUNK, _CHUNK)], sem
            )
            for c in range(n_ch)
        ]
        for cpy in copies:
            cpy.wait()
        pltpu.sync_copy(rows_v, out_hbm.at[pl.ds(wid * b_per_w, b_per_w)])

    return gather_k(time, table)


# speedup vs baseline: 2.0750x; 1.8974x over previous
import functools

import jax
import jax.numpy as jnp
from jax import lax
from jax.experimental import pallas as pl
from jax.experimental.pallas import tpu as pltpu
from jax.experimental.pallas import tpu_sc as plsc

_INFO = plsc.get_sparse_core_info()
_NC = _INFO.num_cores
_NS = _INFO.num_subcores
_NW = _NC * _NS
_CHUNK = 128
_L = 16


def kernel(time, table):
    B = time.shape[0]
    V, D = table.shape
    assert D == _L and B % (_NW * _CHUNK) == 0
    b_per_w = B // _NW
    n_st = (b_per_w * D) // _CHUNK
    spw = b_per_w // _CHUNK

    table_t = table.T.reshape(-1)
    mesh = plsc.VectorSubcoreMesh(core_axis_name="c", subcore_axis_name="s")

    @functools.partial(
        pl.kernel,
        mesh=mesh,
        out_type=jax.ShapeDtypeStruct((D, B), jnp.float32),
        scratch_types=[
            pltpu.VMEM((b_per_w,), jnp.int32),
            pltpu.VMEM((n_st, _CHUNK), jnp.int32),
            pltpu.VMEM((D, b_per_w), jnp.float32),
            pltpu.SemaphoreType.DMA,
            pltpu.SemaphoreType.DMA,
        ],
        compiler_params=pltpu.CompilerParams(use_tc_tiling_on_sc=False),
    )
    def gather_k(time_hbm, table_hbm, out_hbm, idx_v, il_v, o_v, sg, so):
        wid = lax.axis_index("s") * _NC + lax.axis_index("c")
        base = wid * b_per_w
        pltpu.sync_copy(time_hbm.at[pl.ds(base, b_per_w)], idx_v)
        for k in range(D):
            for h in range(spw):
                for t in range(_CHUNK // _L):
                    v = idx_v[pl.ds(h * _CHUNK + t * _L, _L)]
                    il_v[k * spw + h, pl.ds(t * _L, _L)] = v + (k * V)
        copies = []
        for q in range(n_st):
            k, h = q // spw, q % spw
            copies.append(
                pltpu.async_copy(
                    table_hbm.at[il_v.at[q]],
                    o_v.at[k, pl.ds(h * _CHUNK, _CHUNK)],
                    sg,
                )
            )
        for cpy in copies:
            cpy.wait()
        outs = [
            pltpu.async_copy(o_v.at[k], out_hbm.at[k, pl.ds(base, b_per_w)], so)
            for k in range(D)
        ]
        for cpy in outs:
            cpy.wait()

    return gather_k(time, table_t).T
